# baseline (speedup 1.0000x reference)
import functools

import jax
import jax.numpy as jnp
from jax import lax
from jax.experimental import pallas as pl
from jax.experimental.pallas import tpu as pltpu
from jax.experimental.pallas import tpu_sc as plsc

POINTS = 8192
OVERSAMPLE = 3
IMPORTANCE = 0.75
N = POINTS * OVERSAMPLE
K = int(IMPORTANCE * POINTS)
RAND = POINTS - K

PB = 2048
SR, SL = 256, 128
SORT_N = SR * SL
K_ROWS = K // SL


def _make_sc_gather(n_rows_out, n_ch, ch, c):
    mesh = plsc.VectorSubcoreMesh(core_axis_name="c", subcore_axis_name="s")
    info = plsc.get_sparse_core_info()
    nw = info.num_cores * info.num_subcores
    per_w = n_rows_out // nw

    @functools.partial(
        pl.kernel,
        mesh=mesh,
        out_type=jax.ShapeDtypeStruct((n_rows_out, c), jnp.float32),
        scratch_types=[
            pltpu.VMEM((ch,), jnp.int32),
            pltpu.VMEM((ch, c), jnp.float32),
            pltpu.SemaphoreType.DMA,
        ],
    )
    def gather_kernel(table_hbm, idx_hbm, out_hbm, idx_v, rows_v, sem):
        wid = lax.axis_index("s") * info.num_cores + lax.axis_index("c")
        base = wid * per_w

        def body(i, carry):
            off = base + i * ch
            pltpu.sync_copy(idx_hbm.at[pl.ds(off, ch)], idx_v)
            pltpu.async_copy(table_hbm.at[idx_v], rows_v, sem).wait()
            pltpu.sync_copy(rows_v, out_hbm.at[pl.ds(off, ch)])
            return carry

        lax.fori_loop(0, n_ch, body, 0)

    return gather_kernel


def _uncert_body(g_ref, w_ref, o_ref):
    g = g_ref[0, :, :, :96]
    w = w_ref[0, :, 0, :]
    l = (g[0] * w[0][:, None] + g[1] * w[1][:, None]
         + g[2] * w[2][:, None] + g[3] * w[3][:, None])
    m = jnp.max(l, axis=1, keepdims=True)
    e = jnp.exp(l - m)
    t = e[:, 0:8]
    for kk in range(1, 12):
        t = t + e[:, 8 * kk:8 * (kk + 1)]
    t = t[:, :4] + t[:, 4:8]
    t = t[:, :2] + t[:, 2:4]
    s = (t[:, 0] + t[:, 1])[:, None]
    lane = lax.broadcasted_iota(jnp.int32, l.shape, 1)
    nch = l.shape[1]
    first = jnp.min(jnp.where(l == m, lane, nch), axis=1, keepdims=True)
    e2 = jnp.max(jnp.where(lane == first, -1.0, e), axis=1, keepdims=True)
    u = e2 / s - 1.0 / s
    o_ref[0, 0, :] = u[:, 0]


def _roll(x, shift, axis):
    return pltpu.roll(x, shift, axis)


def _partner(x, j, ridx_bit, lidx_bit):
    if j >= SL:
        jr = j // SL
        return jnp.where(ridx_bit, _roll(x, jr, 0), _roll(x, -jr % SR, 0))
    return jnp.where(lidx_bit, _roll(x, j, 1), _roll(x, -j % SL, 1))


def _sort_body(k_ref, a_ref, b_ref, oa_ref, ob_ref):
    key = k_ref[0]
    c0 = a_ref[0]
    c1 = b_ref[0]
    ridx = lax.broadcasted_iota(jnp.int32, (SR, SL), 0)
    lidx = lax.broadcasted_iota(jnp.int32, (SR, SL), 1)
    flat = ridx * SL + lidx
    idx = flat

    kk = 2
    while kk <= SORT_N:
        j = kk // 2
        while j >= 1:
            desc = (flat & kk) == 0
            is_lower = (flat & j) == 0
            if j >= SL:
                bit = (ridx & (j // SL)) != 0
            else:
                bit = (lidx & j) != 0
            kp = _partner(key, j, bit, bit)
            ip = _partner(idx, j, bit, bit)
            ap = _partner(c0, j, bit, bit)
            bp = _partner(c1, j, bit, bit)
            a_first = (key > kp) | ((key == kp) & (idx < ip))
            keep = a_first == (desc == is_lower)
            key = jnp.where(keep, key, kp)
            idx = jnp.where(keep, idx, ip)
            c0 = jnp.where(keep, c0, ap)
            c1 = jnp.where(keep, c1, bp)
            j //= 2
        kk *= 2

    oa_ref[0] = c0[:K_ROWS]
    ob_ref[0] = c1[:K_ROWS]


def kernel(inputs):
    b, h, w_dim, c = inputs.shape
    dtype = inputs.dtype
    rows = (h + 2) * (w_dim + 2)

    key = jax.random.key(42)
    k1, k2 = jax.random.split(key)
    coords = jax.random.uniform(k1, (b, N, 2), dtype=dtype)
    rand_points = jax.random.uniform(k2, (b, RAND, 2), dtype=dtype)

    g = coords[..., ::-1]
    g = (2.0 * g * jnp.asarray([h, w_dim], dtype=dtype) - 1.0) / 2.0
    grid_nw = jnp.floor(g)
    grid_ne = grid_nw + jnp.asarray([1.0, 0.0], dtype=g.dtype)
    grid_sw = grid_nw + jnp.asarray([0.0, 1.0], dtype=g.dtype)
    grid_se = grid_nw + jnp.asarray([1.0, 1.0], dtype=g.dtype)
    w_nw = jnp.prod(grid_se - g, axis=-1)
    w_ne = jnp.prod((grid_sw - g) * jnp.asarray([1.0, -1.0], dtype=g.dtype), axis=-1)
    w_sw = jnp.prod((grid_ne - g) * jnp.asarray([-1.0, 1.0], dtype=g.dtype), axis=-1)
    w_se = jnp.prod(g - grid_nw, axis=-1)

    bounds = jnp.asarray([h + 1, w_dim + 1], dtype=jnp.int32)
    batch_off = (jnp.arange(b, dtype=jnp.int32) * rows)[:, None]

    def flat_idx(corner):
        cc = jnp.clip(corner.astype(jnp.int32) + 1, 0, bounds)
        return batch_off + cc[..., 0] * (w_dim + 2) + cc[..., 1]

    idx4 = jnp.stack([flat_idx(grid_nw), flat_idx(grid_ne),
                      flat_idx(grid_sw), flat_idx(grid_se)], axis=1)
    w4 = jnp.stack([w_nw, w_ne, w_sw, w_se], axis=1)

    nchunk = N // PB
    gb = b * nchunk
    idx_flat = (idx4.reshape(b, 4, nchunk, PB).transpose(0, 2, 1, 3)
                .reshape(gb * 4 * PB))
    wq = (w4.reshape(b, 4, nchunk, PB).transpose(0, 2, 1, 3)
          .reshape(gb, 4, 1, PB))

    cp = 128
    table = jnp.pad(inputs, ((0, 0), (1, 1), (1, 1), (0, cp - c))
                    ).reshape(b * rows, cp)
    ch = 128
    n_rows_out = gb * 4 * PB
    n_ch = n_rows_out // 32 // ch
    gathered = _make_sc_gather(n_rows_out, n_ch, ch, cp)(table, idx_flat)
    gathered = gathered.reshape(gb, 4, PB, cp)

    uncerts = pl.pallas_call(
        _uncert_body,
        grid=(gb,),
        in_specs=[
            pl.BlockSpec((1, 4, PB, cp), lambda i: (i, 0, 0, 0)),
            pl.BlockSpec((1, 4, 1, PB), lambda i: (i, 0, 0, 0)),
        ],
        out_specs=pl.BlockSpec((1, 1, PB), lambda i: (i, 0, 0)),
        out_shape=jax.ShapeDtypeStruct((gb, 1, PB), jnp.float32),
    )(gathered, wq)
    uncerts = uncerts.reshape(b, N)

    pad_rows = SORT_N - N
    keys = jnp.pad(uncerts.reshape(b, N // SL, SL),
                   ((0, 0), (0, pad_rows // SL), (0, 0)),
                   constant_values=-3.0)
    c0p = jnp.pad(coords[..., 0].reshape(b, N // SL, SL),
                  ((0, 0), (0, pad_rows // SL), (0, 0)))
    c1p = jnp.pad(coords[..., 1].reshape(b, N // SL, SL),
                  ((0, 0), (0, pad_rows // SL), (0, 0)))

    c0s, c1s = pl.pallas_call(
        _sort_body,
        grid=(b,),
        in_specs=[pl.BlockSpec((1, SR, SL), lambda i: (i, 0, 0))] * 3,
        out_specs=[pl.BlockSpec((1, K_ROWS, SL), lambda i: (i, 0, 0))] * 2,
        out_shape=[jax.ShapeDtypeStruct((b, K_ROWS, SL), jnp.float32)] * 2,
    )(keys, c0p, c1p)

    top_points = jnp.stack([c0s.reshape(b, K), c1s.reshape(b, K)], axis=-1)
    return jnp.concatenate([top_points, rand_points], axis=1)


# speedup vs baseline: 1.1115x; 1.1115x over previous
import functools

import jax
import jax.numpy as jnp
from jax import lax
from jax.experimental import pallas as pl
from jax.experimental.pallas import tpu as pltpu
from jax.experimental.pallas import tpu_sc as plsc

POINTS = 8192
OVERSAMPLE = 3
IMPORTANCE = 0.75
N = POINTS * OVERSAMPLE
K = int(IMPORTANCE * POINTS)
RAND = POINTS - K

PB = 2048
SR, SL = 256, 128
SORT_N = SR * SL
K_ROWS = K // SL


NBUF = 4


def _make_sc_gather(n_rows_out, n_ch, ch, c):
    mesh = plsc.VectorSubcoreMesh(core_axis_name="c", subcore_axis_name="s")
    info = plsc.get_sparse_core_info()
    nw = info.num_cores * info.num_subcores
    per_w = n_rows_out // nw

    @functools.partial(
        pl.kernel,
        mesh=mesh,
        compiler_params=pltpu.CompilerParams(use_tc_tiling_on_sc=True),
        out_type=jax.ShapeDtypeStruct((n_rows_out, c), jnp.float32),
        scratch_types=(
            [pltpu.VMEM((n_ch, ch), jnp.int32)]
            + [pltpu.VMEM((ch, c), jnp.float32) for _ in range(NBUF)]
            + [pltpu.SemaphoreType.DMA for _ in range(2 * NBUF)]
        ),
    )
    def gather_kernel(table_hbm, idx_hbm, out_hbm, idx_v, *rest):
        bufs = rest[:NBUF]
        gsems = rest[NBUF:2 * NBUF]
        wsems = rest[2 * NBUF:]
        wid = lax.axis_index("s") * info.num_cores + lax.axis_index("c")
        base = wid * per_w
        pltpu.sync_copy(idx_hbm.at[wid], idx_v)

        for j in range(NBUF):
            pltpu.async_copy(table_hbm.at[idx_v.at[j]], bufs[j], gsems[j])

        def body(t, carry):
            for j in range(NBUF):
                cch = t * NBUF + j
                pltpu.make_async_copy(table_hbm.at[idx_v.at[0]], bufs[j],
                                      gsems[j]).wait()
                pltpu.async_copy(
                    bufs[j], out_hbm.at[pl.ds(base + cch * ch, ch)],
                    wsems[j]).wait()

                @pl.when(cch + NBUF < n_ch)
                def _():
                    pltpu.async_copy(table_hbm.at[idx_v.at[(cch + NBUF) % n_ch]],
                                     bufs[j], gsems[j])

            return carry

        lax.fori_loop(0, n_ch // NBUF, body, 0)

    return gather_kernel


def _uncert_body(g_ref, w_ref, o_ref):
    g = g_ref[0, :, :, :96]
    w = w_ref[0, :, 0, :]
    l = (g[0] * w[0][:, None] + g[1] * w[1][:, None]
         + g[2] * w[2][:, None] + g[3] * w[3][:, None])
    m = jnp.max(l, axis=1, keepdims=True)
    e = jnp.exp(l - m)
    t = e[:, 0:8]
    for kk in range(1, 12):
        t = t + e[:, 8 * kk:8 * (kk + 1)]
    t = t[:, :4] + t[:, 4:8]
    t = t[:, :2] + t[:, 2:4]
    s = (t[:, 0] + t[:, 1])[:, None]
    lane = lax.broadcasted_iota(jnp.int32, l.shape, 1)
    nch = l.shape[1]
    first = jnp.min(jnp.where(l == m, lane, nch), axis=1, keepdims=True)
    e2 = jnp.max(jnp.where(lane == first, -1.0, e), axis=1, keepdims=True)
    u = e2 / s - 1.0 / s
    o_ref[0, 0, :] = u[:, 0]


def _roll(x, shift, axis):
    return pltpu.roll(x, shift, axis)


def _partner(x, j, ridx_bit, lidx_bit):
    if j >= SL:
        jr = j // SL
        return jnp.where(ridx_bit, _roll(x, jr, 0), _roll(x, -jr % SR, 0))
    return jnp.where(lidx_bit, _roll(x, j, 1), _roll(x, -j % SL, 1))


def _sort_body(k_ref, a_ref, b_ref, oa_ref, ob_ref):
    key = k_ref[0]
    c0 = a_ref[0]
    c1 = b_ref[0]
    ridx = lax.broadcasted_iota(jnp.int32, (SR, SL), 0)
    lidx = lax.broadcasted_iota(jnp.int32, (SR, SL), 1)
    flat = ridx * SL + lidx
    idx = flat

    kk = 2
    while kk <= SORT_N:
        j = kk // 2
        while j >= 1:
            desc = (flat & kk) == 0
            is_lower = (flat & j) == 0
            if j >= SL:
                bit = (ridx & (j // SL)) != 0
            else:
                bit = (lidx & j) != 0
            kp = _partner(key, j, bit, bit)
            ip = _partner(idx, j, bit, bit)
            ap = _partner(c0, j, bit, bit)
            bp = _partner(c1, j, bit, bit)
            a_first = (key > kp) | ((key == kp) & (idx < ip))
            keep = a_first == (desc == is_lower)
            key = jnp.where(keep, key, kp)
            idx = jnp.where(keep, idx, ip)
            c0 = jnp.where(keep, c0, ap)
            c1 = jnp.where(keep, c1, bp)
            j //= 2
        kk *= 2

    oa_ref[0] = c0[:K_ROWS]
    ob_ref[0] = c1[:K_ROWS]


def kernel(inputs):
    b, h, w_dim, c = inputs.shape
    dtype = inputs.dtype
    rows = (h + 2) * (w_dim + 2)

    key = jax.random.key(42)
    k1, k2 = jax.random.split(key)
    coords = jax.random.uniform(k1, (b, N, 2), dtype=dtype)
    rand_points = jax.random.uniform(k2, (b, RAND, 2), dtype=dtype)

    g = coords[..., ::-1]
    g = (2.0 * g * jnp.asarray([h, w_dim], dtype=dtype) - 1.0) / 2.0
    grid_nw = jnp.floor(g)
    grid_ne = grid_nw + jnp.asarray([1.0, 0.0], dtype=g.dtype)
    grid_sw = grid_nw + jnp.asarray([0.0, 1.0], dtype=g.dtype)
    grid_se = grid_nw + jnp.asarray([1.0, 1.0], dtype=g.dtype)
    w_nw = jnp.prod(grid_se - g, axis=-1)
    w_ne = jnp.prod((grid_sw - g) * jnp.asarray([1.0, -1.0], dtype=g.dtype), axis=-1)
    w_sw = jnp.prod((grid_ne - g) * jnp.asarray([-1.0, 1.0], dtype=g.dtype), axis=-1)
    w_se = jnp.prod(g - grid_nw, axis=-1)

    bounds = jnp.asarray([h + 1, w_dim + 1], dtype=jnp.int32)
    batch_off = (jnp.arange(b, dtype=jnp.int32) * rows)[:, None]

    def flat_idx(corner):
        cc = jnp.clip(corner.astype(jnp.int32) + 1, 0, bounds)
        return batch_off + cc[..., 0] * (w_dim + 2) + cc[..., 1]

    idx4 = jnp.stack([flat_idx(grid_nw), flat_idx(grid_ne),
                      flat_idx(grid_sw), flat_idx(grid_se)], axis=1)
    w4 = jnp.stack([w_nw, w_ne, w_sw, w_se], axis=1)

    nchunk = N // PB
    gb = b * nchunk
    idx_flat = (idx4.reshape(b, 4, nchunk, PB).transpose(0, 2, 1, 3)
                .reshape(gb * 4 * PB))
    wq = (w4.reshape(b, 4, nchunk, PB).transpose(0, 2, 1, 3)
          .reshape(gb, 4, 1, PB))

    cp = 128
    table = jnp.pad(inputs, ((0, 0), (1, 1), (1, 1), (0, cp - c))
                    ).reshape(b * rows, cp)
    ch = 128
    n_rows_out = gb * 4 * PB
    n_ch = n_rows_out // 32 // ch
    idx_w = idx_flat.reshape(32, n_ch, ch)
    gathered = _make_sc_gather(n_rows_out, n_ch, ch, cp)(table, idx_w)
    gathered = gathered.reshape(gb, 4, PB, cp)

    uncerts = pl.pallas_call(
        _uncert_body,
        grid=(gb,),
        in_specs=[
            pl.BlockSpec((1, 4, PB, cp), lambda i: (i, 0, 0, 0)),
            pl.BlockSpec((1, 4, 1, PB), lambda i: (i, 0, 0, 0)),
        ],
        out_specs=pl.BlockSpec((1, 1, PB), lambda i: (i, 0, 0)),
        out_shape=jax.ShapeDtypeStruct((gb, 1, PB), jnp.float32),
    )(gathered, wq)
    uncerts = uncerts.reshape(b, N)

    pad_rows = SORT_N - N
    keys = jnp.pad(uncerts.reshape(b, N // SL, SL),
                   ((0, 0), (0, pad_rows // SL), (0, 0)),
                   constant_values=-3.0)
    c0p = jnp.pad(coords[..., 0].reshape(b, N // SL, SL),
                  ((0, 0), (0, pad_rows // SL), (0, 0)))
    c1p = jnp.pad(coords[..., 1].reshape(b, N // SL, SL),
                  ((0, 0), (0, pad_rows // SL), (0, 0)))

    c0s, c1s = pl.pallas_call(
        _sort_body,
        grid=(b,),
        in_specs=[pl.BlockSpec((1, SR, SL), lambda i: (i, 0, 0))] * 3,
        out_specs=[pl.BlockSpec((1, K_ROWS, SL), lambda i: (i, 0, 0))] * 2,
        out_shape=[jax.ShapeDtypeStruct((b, K_ROWS, SL), jnp.float32)] * 2,
    )(keys, c0p, c1p)

    top_points = jnp.stack([c0s.reshape(b, K), c1s.reshape(b, K)], axis=-1)
    return jnp.concatenate([top_points, rand_points], axis=1)


# speedup vs baseline: 1.2173x; 1.0952x over previous
import functools

import jax
import jax.numpy as jnp
from jax import lax
from jax.experimental import pallas as pl
from jax.experimental.pallas import tpu as pltpu
from jax.experimental.pallas import tpu_sc as plsc

POINTS = 8192
OVERSAMPLE = 3
IMPORTANCE = 0.75
N = POINTS * OVERSAMPLE
K = int(IMPORTANCE * POINTS)
RAND = POINTS - K

PB = 2048
SR, SL = 256, 128
SORT_N = SR * SL
K_ROWS = K // SL


NBUF = 4


def _make_sc_gather(n_rows_out, n_ch, ch, c):
    mesh = plsc.VectorSubcoreMesh(core_axis_name="c", subcore_axis_name="s")
    info = plsc.get_sparse_core_info()
    nw = info.num_cores * info.num_subcores
    per_w = n_rows_out // nw

    @functools.partial(
        pl.kernel,
        mesh=mesh,
        compiler_params=pltpu.CompilerParams(use_tc_tiling_on_sc=True),
        out_type=jax.ShapeDtypeStruct((n_rows_out, c), jnp.float32),
        scratch_types=(
            [pltpu.VMEM((n_ch, ch), jnp.int32)]
            + [pltpu.VMEM((ch, c), jnp.float32) for _ in range(NBUF)]
            + [pltpu.SemaphoreType.DMA for _ in range(2 * NBUF)]
        ),
    )
    def gather_kernel(table_hbm, idx_hbm, out_hbm, idx_v, *rest):
        bufs = rest[:NBUF]
        gsems = rest[NBUF:2 * NBUF]
        wsems = rest[2 * NBUF:]
        wid = lax.axis_index("s") * info.num_cores + lax.axis_index("c")
        base = wid * per_w
        pltpu.sync_copy(idx_hbm.at[wid], idx_v)

        for j in range(NBUF):
            pltpu.async_copy(table_hbm.at[idx_v.at[j]], bufs[j], gsems[j])

        def body(t, carry):
            for j in range(NBUF):
                cch = t * NBUF + j
                pltpu.make_async_copy(table_hbm.at[idx_v.at[0]], bufs[j],
                                      gsems[j]).wait()
                pltpu.async_copy(
                    bufs[j], out_hbm.at[pl.ds(base + cch * ch, ch)],
                    wsems[j]).wait()

                @pl.when(cch + NBUF < n_ch)
                def _():
                    pltpu.async_copy(table_hbm.at[idx_v.at[(cch + NBUF) % n_ch]],
                                     bufs[j], gsems[j])

            return carry

        lax.fori_loop(0, n_ch // NBUF, body, 0)

    return gather_kernel


def _uncert_body(g_ref, w_ref, o_ref):
    g = g_ref[0, :, :, :96]
    w = w_ref[0, :, 0, :]
    l = (g[0] * w[0][:, None] + g[1] * w[1][:, None]
         + g[2] * w[2][:, None] + g[3] * w[3][:, None])
    m = jnp.max(l, axis=1, keepdims=True)
    e = jnp.exp(l - m)
    t = e[:, 0:8]
    for kk in range(1, 12):
        t = t + e[:, 8 * kk:8 * (kk + 1)]
    t = t[:, :4] + t[:, 4:8]
    t = t[:, :2] + t[:, 2:4]
    s = (t[:, 0] + t[:, 1])[:, None]
    lane = lax.broadcasted_iota(jnp.int32, l.shape, 1)
    nch = l.shape[1]
    first = jnp.min(jnp.where(l == m, lane, nch), axis=1, keepdims=True)
    e2 = jnp.max(jnp.where(lane == first, -1.0, e), axis=1, keepdims=True)
    u = e2 / s - 1.0 / s
    o_ref[0, 0, :] = u[:, 0]


def _roll(x, shift, axis):
    return pltpu.roll(x, shift, axis)


def _partner(x, j, ridx_bit, lidx_bit):
    if j >= SL:
        jr = j // SL
        return jnp.where(ridx_bit, _roll(x, jr, 0), _roll(x, -jr % SR, 0))
    return jnp.where(lidx_bit, _roll(x, j, 1), _roll(x, -j % SL, 1))


def _sort_body(k_ref, a_ref, b_ref, oa_ref, ob_ref):
    key = k_ref[0]
    c0 = a_ref[0]
    c1 = b_ref[0]
    ridx = lax.broadcasted_iota(jnp.int32, (SR, SL), 0)
    lidx = lax.broadcasted_iota(jnp.int32, (SR, SL), 1)
    flat = ridx * SL + lidx
    idx = flat

    kk = 2
    while kk <= SORT_N:
        j = kk // 2
        while j >= 1:
            desc = (flat & kk) == 0
            is_lower = (flat & j) == 0
            if j >= SL:
                bit = (ridx & (j // SL)) != 0
            else:
                bit = (lidx & j) != 0
            kp = _partner(key, j, bit, bit)
            ip = _partner(idx, j, bit, bit)
            ap = _partner(c0, j, bit, bit)
            bp = _partner(c1, j, bit, bit)
            a_first = (key > kp) | ((key == kp) & (idx < ip))
            keep = a_first == (desc == is_lower)
            key = jnp.where(keep, key, kp)
            idx = jnp.where(keep, idx, ip)
            c0 = jnp.where(keep, c0, ap)
            c1 = jnp.where(keep, c1, bp)
            j //= 2
        kk *= 2

    oa_ref[0] = c0[:K_ROWS]
    ob_ref[0] = c1[:K_ROWS]


def kernel(inputs):
    b, h, w_dim, c = inputs.shape
    dtype = inputs.dtype
    rows = (h + 2) * (w_dim + 2)

    key = jax.random.key(42)
    k1, k2 = jax.random.split(key)
    coords = jax.random.uniform(k1, (b, N, 2), dtype=dtype)
    rand_points = jax.random.uniform(k2, (b, RAND, 2), dtype=dtype)

    g = coords[..., ::-1]
    g = (2.0 * g * jnp.asarray([h, w_dim], dtype=dtype) - 1.0) / 2.0
    grid_nw = jnp.floor(g)
    grid_ne = grid_nw + jnp.asarray([1.0, 0.0], dtype=g.dtype)
    grid_sw = grid_nw + jnp.asarray([0.0, 1.0], dtype=g.dtype)
    grid_se = grid_nw + jnp.asarray([1.0, 1.0], dtype=g.dtype)
    w_nw = jnp.prod(grid_se - g, axis=-1)
    w_ne = jnp.prod((grid_sw - g) * jnp.asarray([1.0, -1.0], dtype=g.dtype), axis=-1)
    w_sw = jnp.prod((grid_ne - g) * jnp.asarray([-1.0, 1.0], dtype=g.dtype), axis=-1)
    w_se = jnp.prod(g - grid_nw, axis=-1)

    bounds = jnp.asarray([h + 1, w_dim + 1], dtype=jnp.int32)
    batch_off = (jnp.arange(b, dtype=jnp.int32) * rows)[:, None]

    def flat_idx(corner):
        cc = jnp.clip(corner.astype(jnp.int32) + 1, 0, bounds)
        return batch_off + cc[..., 0] * (w_dim + 2) + cc[..., 1]

    idx4 = jnp.stack([flat_idx(grid_nw), flat_idx(grid_ne),
                      flat_idx(grid_sw), flat_idx(grid_se)], axis=1)
    w4 = jnp.stack([w_nw, w_ne, w_sw, w_se], axis=1)

    nchunk = N // PB
    gb = b * nchunk
    idx_flat = (idx4.reshape(b, 4, nchunk, PB).transpose(0, 2, 1, 3)
                .reshape(gb * 4 * PB))
    wq = (w4.reshape(b, 4, nchunk, PB).transpose(0, 2, 1, 3)
          .reshape(gb, 4, 1, PB))

    cp = 128
    table = jnp.pad(inputs, ((0, 0), (1, 1), (1, 1), (0, cp - c))
                    ).reshape(b * rows, cp)
    ch = 128
    nsplit = 4
    gbs = gb // nsplit
    n_rows_s = gbs * 4 * PB
    n_ch = n_rows_s // 32 // ch
    unc_parts = []
    for si in range(nsplit):
        idx_s = lax.slice_in_dim(idx_flat, si * n_rows_s, (si + 1) * n_rows_s
                                 ).reshape(32, n_ch, ch)
        gathered = _make_sc_gather(n_rows_s, n_ch, ch, cp)(table, idx_s)
        gathered = gathered.reshape(gbs, 4, PB, cp)
        wq_s = lax.slice_in_dim(wq, si * gbs, (si + 1) * gbs)
        unc_parts.append(pl.pallas_call(
            _uncert_body,
            grid=(gbs,),
            in_specs=[
                pl.BlockSpec((1, 4, PB, cp), lambda i: (i, 0, 0, 0)),
                pl.BlockSpec((1, 4, 1, PB), lambda i: (i, 0, 0, 0)),
            ],
            out_specs=pl.BlockSpec((1, 1, PB), lambda i: (i, 0, 0)),
            out_shape=jax.ShapeDtypeStruct((gbs, 1, PB), jnp.float32),
        )(gathered, wq_s))
    uncerts = jnp.concatenate(unc_parts, axis=0).reshape(b, N)

    pad_rows = SORT_N - N
    keys = jnp.pad(uncerts.reshape(b, N // SL, SL),
                   ((0, 0), (0, pad_rows // SL), (0, 0)),
                   constant_values=-3.0)
    c0p = jnp.pad(coords[..., 0].reshape(b, N // SL, SL),
                  ((0, 0), (0, pad_rows // SL), (0, 0)))
    c1p = jnp.pad(coords[..., 1].reshape(b, N // SL, SL),
                  ((0, 0), (0, pad_rows // SL), (0, 0)))

    c0s, c1s = pl.pallas_call(
        _sort_body,
        grid=(b,),
        in_specs=[pl.BlockSpec((1, SR, SL), lambda i: (i, 0, 0))] * 3,
        out_specs=[pl.BlockSpec((1, K_ROWS, SL), lambda i: (i, 0, 0))] * 2,
        out_shape=[jax.ShapeDtypeStruct((b, K_ROWS, SL), jnp.float32)] * 2,
    )(keys, c0p, c1p)

    top_points = jnp.stack([c0s.reshape(b, K), c1s.reshape(b, K)], axis=-1)
    return jnp.concatenate([top_points, rand_points], axis=1)


# speedup vs baseline: 1.3423x; 1.1027x over previous
import functools

import jax
import jax.numpy as jnp
from jax import lax
from jax.experimental import pallas as pl
from jax.experimental.pallas import tpu as pltpu
from jax.experimental.pallas import tpu_sc as plsc

POINTS = 8192
OVERSAMPLE = 3
IMPORTANCE = 0.75
N = POINTS * OVERSAMPLE
K = int(IMPORTANCE * POINTS)
RAND = POINTS - K

PB = 2048
SR, SL = 256, 128
SORT_N = SR * SL
K_ROWS = K // SL


NBUF = 4


def _make_sc_gather(n_rows_out, n_ch, ch, c):
    mesh = plsc.VectorSubcoreMesh(core_axis_name="c", subcore_axis_name="s")
    info = plsc.get_sparse_core_info()
    nw = info.num_cores * info.num_subcores
    per_w = n_rows_out // nw

    @functools.partial(
        pl.kernel,
        mesh=mesh,
        compiler_params=pltpu.CompilerParams(use_tc_tiling_on_sc=True),
        out_type=jax.ShapeDtypeStruct((n_rows_out, c), jnp.float32),
        scratch_types=(
            [pltpu.VMEM((n_ch, ch), jnp.int32)]
            + [pltpu.VMEM((ch, c), jnp.float32) for _ in range(NBUF)]
            + [pltpu.SemaphoreType.DMA for _ in range(2 * NBUF)]
        ),
    )
    def gather_kernel(table_hbm, idx_hbm, out_hbm, idx_v, *rest):
        bufs = rest[:NBUF]
        gsems = rest[NBUF:2 * NBUF]
        wsems = rest[2 * NBUF:]
        wid = lax.axis_index("s") * info.num_cores + lax.axis_index("c")
        base = wid * per_w
        pltpu.sync_copy(idx_hbm.at[wid], idx_v)

        for j in range(NBUF):
            pltpu.async_copy(table_hbm.at[idx_v.at[j]], bufs[j], gsems[j])

        def body(t, carry):
            for j in range(NBUF):
                cch = t * NBUF + j
                pltpu.make_async_copy(table_hbm.at[idx_v.at[0]], bufs[j],
                                      gsems[j]).wait()
                pltpu.async_copy(
                    bufs[j], out_hbm.at[pl.ds(base + cch * ch, ch)],
                    wsems[j]).wait()

                @pl.when(cch + NBUF < n_ch)
                def _():
                    pltpu.async_copy(table_hbm.at[idx_v.at[(cch + NBUF) % n_ch]],
                                     bufs[j], gsems[j])

            return carry

        lax.fori_loop(0, n_ch // NBUF, body, 0)

    return gather_kernel


def _uncert_body(g_ref, w_ref, o_ref):
    g = g_ref[0, :, :, :96]
    w = w_ref[0, :, 0, :]
    l = (g[0] * w[0][:, None] + g[1] * w[1][:, None]
         + g[2] * w[2][:, None] + g[3] * w[3][:, None])
    m = jnp.max(l, axis=1, keepdims=True)
    e = jnp.exp(l - m)
    t = e[:, 0:8]
    for kk in range(1, 12):
        t = t + e[:, 8 * kk:8 * (kk + 1)]
    t = t[:, :4] + t[:, 4:8]
    t = t[:, :2] + t[:, 2:4]
    s = (t[:, 0] + t[:, 1])[:, None]
    lane = lax.broadcasted_iota(jnp.int32, l.shape, 1)
    nch = l.shape[1]
    first = jnp.min(jnp.where(l == m, lane, nch), axis=1, keepdims=True)
    e2 = jnp.max(jnp.where(lane == first, -1.0, e), axis=1, keepdims=True)
    u = e2 / s - 1.0 / s
    o_ref[0, 0, :] = u[:, 0]


def _roll(x, shift, axis):
    return pltpu.roll(x, shift, axis)


def _partner(x, j, ridx_bit, lidx_bit):
    if j >= SL:
        jr = j // SL
        return jnp.where(ridx_bit, _roll(x, jr, 0), _roll(x, -jr % SR, 0))
    return jnp.where(lidx_bit, _roll(x, j, 1), _roll(x, -j % SL, 1))


def _sort_body(k_ref, a_ref, b_ref, oa_ref, ob_ref):
    key = k_ref[0]
    c0 = a_ref[0]
    c1 = b_ref[0]
    ridx = lax.broadcasted_iota(jnp.int32, (SR, SL), 0)
    lidx = lax.broadcasted_iota(jnp.int32, (SR, SL), 1)
    flat = ridx * SL + lidx
    idx = flat

    kk = 2
    while kk <= SORT_N:
        j = kk // 2
        while j >= 1:
            desc = (flat & kk) == 0
            is_lower = (flat & j) == 0
            if j >= SL:
                bit = (ridx & (j // SL)) != 0
            else:
                bit = (lidx & j) != 0
            kp = _partner(key, j, bit, bit)
            ip = _partner(idx, j, bit, bit)
            ap = _partner(c0, j, bit, bit)
            bp = _partner(c1, j, bit, bit)
            a_first = (key > kp) | ((key == kp) & (idx < ip))
            keep = a_first == (desc == is_lower)
            key = jnp.where(keep, key, kp)
            idx = jnp.where(keep, idx, ip)
            c0 = jnp.where(keep, c0, ap)
            c1 = jnp.where(keep, c1, bp)
            j //= 2
        kk *= 2

    oa_ref[0] = c0[:K_ROWS]
    ob_ref[0] = c1[:K_ROWS]


def kernel(inputs):
    b, h, w_dim, c = inputs.shape
    dtype = inputs.dtype
    rows = (h + 2) * (w_dim + 2)

    key = jax.random.key(42)
    k1, k2 = jax.random.split(key)
    coords = jax.random.uniform(k1, (b, N, 2), dtype=dtype)
    rand_points = jax.random.uniform(k2, (b, RAND, 2), dtype=dtype)

    g = coords[..., ::-1]
    g = (2.0 * g * jnp.asarray([h, w_dim], dtype=dtype) - 1.0) / 2.0
    grid_nw = jnp.floor(g)
    grid_ne = grid_nw + jnp.asarray([1.0, 0.0], dtype=g.dtype)
    grid_sw = grid_nw + jnp.asarray([0.0, 1.0], dtype=g.dtype)
    grid_se = grid_nw + jnp.asarray([1.0, 1.0], dtype=g.dtype)
    w_nw = jnp.prod(grid_se - g, axis=-1)
    w_ne = jnp.prod((grid_sw - g) * jnp.asarray([1.0, -1.0], dtype=g.dtype), axis=-1)
    w_sw = jnp.prod((grid_ne - g) * jnp.asarray([-1.0, 1.0], dtype=g.dtype), axis=-1)
    w_se = jnp.prod(g - grid_nw, axis=-1)

    bounds = jnp.asarray([h + 1, w_dim + 1], dtype=jnp.int32)
    batch_off = (jnp.arange(b, dtype=jnp.int32) * rows)[:, None]

    def flat_idx(corner):
        cc = jnp.clip(corner.astype(jnp.int32) + 1, 0, bounds)
        return batch_off + cc[..., 0] * (w_dim + 2) + cc[..., 1]

    idx4 = jnp.stack([flat_idx(grid_nw), flat_idx(grid_ne),
                      flat_idx(grid_sw), flat_idx(grid_se)], axis=1)
    w4 = jnp.stack([w_nw, w_ne, w_sw, w_se], axis=1)

    nchunk = N // PB
    gb = b * nchunk
    idx_flat = (idx4.reshape(b, 4, nchunk, PB).transpose(0, 2, 1, 3)
                .reshape(gb * 4 * PB))
    wq = (w4.reshape(b, 4, nchunk, PB).transpose(0, 2, 1, 3)
          .reshape(gb, 4, 1, PB))

    cp = 128
    ch = 128
    nsplit = 4
    bs = b // nsplit
    gbs = gb // nsplit
    n_rows_s = gbs * 4 * PB
    n_ch = n_rows_s // 32 // ch
    unc_parts = []
    for si in range(nsplit):
        table = jnp.pad(lax.slice_in_dim(inputs, si * bs, (si + 1) * bs),
                        ((0, 0), (1, 1), (1, 1), (0, cp - c))
                        ).reshape(bs * rows, cp)
        idx_s = (lax.slice_in_dim(idx_flat, si * n_rows_s, (si + 1) * n_rows_s)
                 - jnp.int32(si * bs * rows)).reshape(32, n_ch, ch)
        gathered = _make_sc_gather(n_rows_s, n_ch, ch, cp)(table, idx_s)
        gathered = gathered.reshape(gbs, 4, PB, cp)
        wq_s = lax.slice_in_dim(wq, si * gbs, (si + 1) * gbs)
        unc_parts.append(pl.pallas_call(
            _uncert_body,
            grid=(gbs,),
            in_specs=[
                pl.BlockSpec((1, 4, PB, cp), lambda i: (i, 0, 0, 0)),
                pl.BlockSpec((1, 4, 1, PB), lambda i: (i, 0, 0, 0)),
            ],
            out_specs=pl.BlockSpec((1, 1, PB), lambda i: (i, 0, 0)),
            out_shape=jax.ShapeDtypeStruct((gbs, 1, PB), jnp.float32),
        )(gathered, wq_s))
    uncerts = jnp.concatenate(unc_parts, axis=0).reshape(b, N)

    pad_rows = SORT_N - N
    keys = jnp.pad(uncerts.reshape(b, N // SL, SL),
                   ((0, 0), (0, pad_rows // SL), (0, 0)),
                   constant_values=-3.0)
    c0p = jnp.pad(coords[..., 0].reshape(b, N // SL, SL),
                  ((0, 0), (0, pad_rows // SL), (0, 0)))
    c1p = jnp.pad(coords[..., 1].reshape(b, N // SL, SL),
                  ((0, 0), (0, pad_rows // SL), (0, 0)))

    c0s, c1s = pl.pallas_call(
        _sort_body,
        grid=(b,),
        in_specs=[pl.BlockSpec((1, SR, SL), lambda i: (i, 0, 0))] * 3,
        out_specs=[pl.BlockSpec((1, K_ROWS, SL), lambda i: (i, 0, 0))] * 2,
        out_shape=[jax.ShapeDtypeStruct((b, K_ROWS, SL), jnp.float32)] * 2,
    )(keys, c0p, c1p)

    top_points = jnp.stack([c0s.reshape(b, K), c1s.reshape(b, K)], axis=-1)
    return jnp.concatenate([top_points, rand_points], axis=1)
